# baseline (speedup 1.0000x reference)
import functools

import jax
import jax.numpy as jnp
from jax import lax
from jax.experimental import pallas as pl
from jax.experimental.pallas import tpu as pltpu
from jax.experimental.pallas import tpu_sc as plsc

_HIDDEN = 768
_BATCH, _N_NEWS, _SIG_LEN = 16, 32, 64
_B = _BATCH * _N_NEWS * _SIG_LEN
_NC, _NS = 2, 16
_NW = _NC * _NS
_BPW = _B // _NW
_NPW = _BPW // _SIG_LEN
_CHUNK = 64
_NCHUNK = _BPW // _CHUNK
_NBUF = 2
_NGRP = _NCHUNK // _NBUF

_mesh = plsc.VectorSubcoreMesh(core_axis_name="c", subcore_axis_name="s")


@functools.partial(
    pl.kernel,
    out_type=jax.ShapeDtypeStruct((_B, _HIDDEN), jnp.float32),
    mesh=_mesh,
    scratch_types=[
        pltpu.VMEM((_NPW, _SIG_LEN), jnp.int32),
        pltpu.VMEM((_NBUF, _CHUNK, _HIDDEN), jnp.float32),
        [pltpu.SemaphoreType.DMA] * _NBUF,
        [pltpu.SemaphoreType.DMA] * _NBUF,
    ],
)
def _emb_lookup(idx_hbm, table_hbm, out_hbm, idx_v, rows_v, gsems, ssems):
    wid = lax.axis_index("s") * _NC + lax.axis_index("c")
    base = wid * _BPW
    batch = wid // 2
    half = wid % 2

    idx_flat = idx_v

    def gather(b, c):
        pltpu.make_async_copy(
            table_hbm.at[idx_flat.at[c]], rows_v.at[b], gsems[b]
        ).start()

    def gather_wait(b):
        pltpu.make_async_copy(
            table_hbm.at[idx_flat.at[0]], rows_v.at[b], gsems[b]
        ).wait()

    def scatter(b, c):
        pltpu.make_async_copy(
            rows_v.at[b], out_hbm.at[pl.ds(base + c * _CHUNK, _CHUNK)], ssems[b]
        ).start()

    def scatter_wait(b):
        pltpu.make_async_copy(
            rows_v.at[b], out_hbm.at[pl.ds(base, _CHUNK)], ssems[b]
        ).wait()

    pltpu.sync_copy(idx_hbm.at[batch, pl.ds(half * _NPW, _NPW)], idx_v)
    for b in range(_NBUF):
        gather(b, b)

    def body(g, _):
        for b in range(_NBUF):
            c = g * _NBUF + b
            gather_wait(b)
            scatter(b, c)
            scatter_wait(b)
            gather(b, c + _NBUF)
        return ()

    lax.fori_loop(0, _NGRP - 1, body, (), unroll=False)

    for b in range(_NBUF):
        c = (_NGRP - 1) * _NBUF + b
        gather_wait(b)
        scatter(b, c)
    for b in range(_NBUF):
        scatter_wait(b)


def kernel(news_batch, word_embedding):
    out = _emb_lookup(news_batch, word_embedding)
    return out.reshape(_BATCH, _N_NEWS, _SIG_LEN, _HIDDEN)


# speedup vs baseline: 1.0031x; 1.0031x over previous
import functools

import jax
import jax.numpy as jnp
from jax import lax
from jax.experimental import pallas as pl
from jax.experimental.pallas import tpu as pltpu
from jax.experimental.pallas import tpu_sc as plsc

_HIDDEN = 768
_BATCH, _N_NEWS, _SIG_LEN = 16, 32, 64
_B = _BATCH * _N_NEWS * _SIG_LEN
_NC, _NS = 2, 16
_NW = _NC * _NS
_BPW = _B // _NW
_CHUNK = 16
_NCHUNK = _BPW // _CHUNK
_NBUF = 4
_NGRP = _NCHUNK // _NBUF

_mesh = plsc.VectorSubcoreMesh(core_axis_name="c", subcore_axis_name="s")


@functools.partial(
    pl.kernel,
    out_type=jax.ShapeDtypeStruct((_B, _HIDDEN), jnp.float32),
    mesh=_mesh,
    scratch_types=[
        pltpu.VMEM((_NCHUNK, _CHUNK), jnp.int32),
        pltpu.VMEM((_NBUF, _CHUNK, _HIDDEN), jnp.float32),
        [pltpu.SemaphoreType.DMA] * _NBUF,
        [pltpu.SemaphoreType.DMA] * _NBUF,
    ],
)
def _emb_lookup(idx_hbm, table_hbm, out_hbm, idx_v, rows_v, gsems, ssems):
    wid = lax.axis_index("s") * _NC + lax.axis_index("c")
    base = wid * _BPW

    def gather(b, c):
        pltpu.make_async_copy(
            table_hbm.at[idx_v.at[c]], rows_v.at[b], gsems[b]
        ).start()

    def gather_wait(b):
        pltpu.make_async_copy(
            table_hbm.at[idx_v.at[0]], rows_v.at[b], gsems[b]
        ).wait()

    def scatter(b, c):
        pltpu.make_async_copy(
            rows_v.at[b], out_hbm.at[pl.ds(base + c * _CHUNK, _CHUNK)], ssems[b]
        ).start()

    def scatter_wait(b):
        pltpu.make_async_copy(
            rows_v.at[b], out_hbm.at[pl.ds(base, _CHUNK)], ssems[b]
        ).wait()

    pltpu.sync_copy(idx_hbm.at[wid], idx_v)
    for b in range(_NBUF):
        gather(b, b)

    def body(g, _):
        for b in range(_NBUF):
            c = g * _NBUF + b
            gather_wait(b)
            scatter(b, c)
            scatter_wait(b)
            gather(b, c + _NBUF)
        return ()

    lax.fori_loop(0, _NGRP - 1, body, (), unroll=False)

    for b in range(_NBUF):
        c = (_NGRP - 1) * _NBUF + b
        gather_wait(b)
        scatter(b, c)
    for b in range(_NBUF):
        scatter_wait(b)


def kernel(news_batch, word_embedding):
    idx = news_batch.reshape(_NW, _NCHUNK, _CHUNK)
    out = _emb_lookup(idx, word_embedding)
    return out.reshape(_BATCH, _N_NEWS, _SIG_LEN, _HIDDEN)


# speedup vs baseline: 1.0094x; 1.0063x over previous
import functools

import jax
import jax.numpy as jnp
from jax import lax
from jax.experimental import pallas as pl
from jax.experimental.pallas import tpu as pltpu
from jax.experimental.pallas import tpu_sc as plsc

_HIDDEN = 768
_BATCH, _N_NEWS, _SIG_LEN = 16, 32, 64
_B = _BATCH * _N_NEWS * _SIG_LEN
_NC, _NS = 2, 16
_NW = _NC * _NS
_BPW = _B // _NW
_CHUNK = 32
_NCHUNK = _BPW // _CHUNK
_NBUF = 4
_NGRP = _NCHUNK // _NBUF

_mesh = plsc.VectorSubcoreMesh(core_axis_name="c", subcore_axis_name="s")


@functools.partial(
    pl.kernel,
    out_type=jax.ShapeDtypeStruct((_B, _HIDDEN), jnp.float32),
    mesh=_mesh,
    scratch_types=[
        pltpu.VMEM((_NCHUNK, _CHUNK), jnp.int32),
        pltpu.VMEM((_NBUF, _CHUNK, _HIDDEN), jnp.float32),
        [pltpu.SemaphoreType.DMA] * _NBUF,
        [pltpu.SemaphoreType.DMA] * _NBUF,
    ],
)
def _emb_lookup(idx_hbm, table_hbm, out_hbm, idx_v, rows_v, gsems, ssems):
    wid = lax.axis_index("s") * _NC + lax.axis_index("c")
    base = wid * _BPW

    def gather(b, c):
        pltpu.make_async_copy(
            table_hbm.at[idx_v.at[c]], rows_v.at[b], gsems[b]
        ).start()

    def gather_wait(b):
        pltpu.make_async_copy(
            table_hbm.at[idx_v.at[0]], rows_v.at[b], gsems[b]
        ).wait()

    def scatter(b, c):
        pltpu.make_async_copy(
            rows_v.at[b], out_hbm.at[pl.ds(base + c * _CHUNK, _CHUNK)], ssems[b]
        ).start()

    def scatter_wait(b):
        pltpu.make_async_copy(
            rows_v.at[b], out_hbm.at[pl.ds(base, _CHUNK)], ssems[b]
        ).wait()

    pltpu.sync_copy(idx_hbm.at[wid], idx_v)
    for b in range(_NBUF):
        gather(b, b)

    def body(g, _):
        for b in range(_NBUF):
            c = g * _NBUF + b
            gather_wait(b)
            scatter(b, c)
            scatter_wait(b)
            gather(b, c + _NBUF)
        return ()

    lax.fori_loop(0, _NGRP - 1, body, (), unroll=False)

    for b in range(_NBUF):
        c = (_NGRP - 1) * _NBUF + b
        gather_wait(b)
        scatter(b, c)
    for b in range(_NBUF):
        scatter_wait(b)


def kernel(news_batch, word_embedding):
    idx = news_batch.reshape(_NW, _NCHUNK, _CHUNK)
    out = _emb_lookup(idx, word_embedding)
    return out.reshape(_BATCH, _N_NEWS, _SIG_LEN, _HIDDEN)
